# baseline (speedup 1.0000x reference)
import functools

import jax
import jax.numpy as jnp
from jax import lax
from jax.experimental import pallas as pl
from jax.experimental.pallas import tpu as pltpu
from jax.experimental.pallas import tpu_sc as plsc


F = 128
NS = 16
NC = 2
EPS = 1e-5


def _layer_norm(x, g, b):
    mu = jnp.mean(x, axis=-1, keepdims=True)
    var = jnp.mean((x - mu) ** 2, axis=-1, keepdims=True)
    return (x - mu) / jnp.sqrt(var + EPS) * g + b


def _make_segsum(C, N, EP):
    P = C // NC
    assert N % (NS * 8) == 0
    rows_per_sub = N // NS
    erows = EP // F
    assert erows % (NS * 8) == 0
    erows_per_sub = erows // NS
    npad = N + NS

    mesh = plsc.VectorSubcoreMesh(core_axis_name="c", subcore_axis_name="s",
                                  num_cores=NC, num_subcores=NS)

    NBUF = 2
    IB = 40
    UNROLL = 4
    assert erows_per_sub % IB == 0 and IB % (NBUF * UNROLL) == 0

    @functools.partial(
        pl.kernel,
        mesh=mesh,
        out_type=jax.ShapeDtypeStruct((C, N, F), jnp.float32),
        scratch_types=[
            pltpu.VMEM((IB, F), jnp.int32),
            pltpu.VMEM((IB, F), jnp.int32),
        ] + [pltpu.VMEM((F, F), jnp.float32) for _ in range(NBUF)]
          + [pltpu.SemaphoreType.DMA for _ in range(2 * NBUF)]
          + [pltpu.VMEM_SHARED((npad, F), jnp.float32)],
    )
    def segsum(x_flat, srcC, dst2, out, src_v, dst_v, *rest):
        bufs = rest[:NBUF]
        gsem = rest[NBUF:2 * NBUF]
        ssem = rest[2 * NBUF:3 * NBUF]
        agg = rest[3 * NBUF]
        core = lax.axis_index("c")
        sid = lax.axis_index("s")
        r0 = sid * rows_per_sub
        e0 = sid * erows_per_sub

        def gather(b, j):
            pltpu.async_copy(x_flat.at[src_v.at[b]], bufs[j], gsem[j])

        def gather_wait(b, j):
            pltpu.make_async_copy(x_flat.at[src_v.at[b]], bufs[j],
                                  gsem[j]).wait()

        def scatter(b, j):
            pltpu.async_copy(bufs[j], agg.at[dst_v.at[b]], ssem[j], add=True)

        def scatter_wait(b, j):
            pltpu.make_async_copy(bufs[j], agg.at[dst_v.at[b]],
                                  ssem[j]).wait()

        nblocks = erows_per_sub // IB

        def load_idx(c, t):
            pltpu.sync_copy(srcC.at[c, pl.ds(e0 + t * IB, IB)], src_v)
            pltpu.sync_copy(dst2.at[pl.ds(e0 + t * IB, IB)], dst_v)

        load_idx(core * P, 0)
        for j in range(NBUF):
            gather(j, j)
        for p in range(P):
            c = core * P + p
            pltpu.sync_copy(x_flat.at[pl.ds(c * N + r0, rows_per_sub)],
                            agg.at[pl.ds(r0, rows_per_sub)])
            plsc.subcore_barrier()

            def idx_block(t, carry):
                def edge_group(g, carry2):
                    for u in range(UNROLL):
                        for j in range(NBUF):
                            b = g * (NBUF * UNROLL) + u * NBUF + j
                            gather_wait(b, j)
                            scatter(b, j)

                            @pl.when(b + NBUF < IB)
                            def _():
                                scatter_wait(b, j)
                                gather(b + NBUF, j)
                    return carry2

                lax.fori_loop(0, IB // (NBUF * UNROLL), edge_group, 0)
                for j in range(NBUF):
                    scatter_wait(IB - NBUF + j, j)

                @pl.when(t + 1 < nblocks)
                def _():
                    load_idx(c, t + 1)
                    for j in range(NBUF):
                        gather(j, j)
                return carry

            lax.fori_loop(0, nblocks, idx_block, 0)
            if p + 1 < P:
                load_idx(c + 1, 0)
                for j in range(NBUF):
                    gather(j, j)
            plsc.subcore_barrier()
            pltpu.sync_copy(agg.at[pl.ds(r0, rows_per_sub)],
                            out.at[c, pl.ds(r0, rows_per_sub)])

    return segsum


def _make_input_ln(N, NP, D, R):
    C = D // F
    grid = (pl.cdiv(NP, R),)

    def body(h_ref, g_ref, b_ref, out_ref):
        x = _layer_norm(h_ref[...], g_ref[...], b_ref[...])
        row = pl.program_id(0) * R + lax.broadcasted_iota(jnp.int32, (R, 1), 0)
        x = jnp.where(row < N, x, 0.0)
        for c in range(C):
            out_ref[c] = x[:, c * F:(c + 1) * F]

    return pl.pallas_call(
        body,
        grid=grid,
        in_specs=[
            pl.BlockSpec((R, D), lambda i: (i, 0)),
            pl.BlockSpec((1, D), lambda i: (0, 0)),
            pl.BlockSpec((1, D), lambda i: (0, 0)),
        ],
        out_specs=pl.BlockSpec((C, R, F), lambda i: (0, i, 0)),
        out_shape=jax.ShapeDtypeStruct((C, NP, F), jnp.float32),
    )


def _make_mlp(N, NP, C_in, H, R, has_res, is_last):
    C_out = H // F
    grid = (pl.cdiv(NP, R),)

    def body(*refs):
        if has_res:
            (z_ref, x_ref, w1_ref, b1_ref, g1_ref, bn1_ref, w2_ref, b2_ref,
             g2_ref, bn2_ref, gn_ref, bnn_ref, out_ref) = refs
        else:
            (z_ref, w1_ref, b1_ref, g1_ref, bn1_ref, w2_ref, b2_ref,
             g2_ref, bn2_ref, gn_ref, bnn_ref, out_ref) = refs
            x_ref = None
        acc = jnp.zeros((R, H), jnp.float32)
        for c in range(C_in):
            acc = acc + jnp.dot(z_ref[c], w1_ref[c * F:(c + 1) * F, :],
                                preferred_element_type=jnp.float32)
        z = acc + b1_ref[...]
        z = jnp.maximum(_layer_norm(z, g1_ref[...], bn1_ref[...]), 0.0)
        z = jnp.dot(z, w2_ref[...], preferred_element_type=jnp.float32)
        z = z + b2_ref[...]
        z = jnp.maximum(_layer_norm(z, g2_ref[...], bn2_ref[...]), 0.0)
        z = jnp.maximum(_layer_norm(z, gn_ref[...], bnn_ref[...]), 0.0)
        if has_res:
            z = z + jnp.concatenate([x_ref[c] for c in range(C_in)], axis=-1)
        if is_last:
            i = pl.program_id(0)
            row = i * R + lax.broadcasted_iota(jnp.int32, (R, 1), 0)
            zm = jnp.where(row < N, z, 0.0)

            @pl.when(i == 0)
            def _():
                out_ref[...] = jnp.zeros_like(out_ref)

            out_ref[...] += jnp.sum(zm, axis=0, keepdims=True) / N
        else:
            for c in range(C_out):
                out_ref[c] = z[:, c * F:(c + 1) * F]

    vec = lambda: pl.BlockSpec((1, H), lambda i: (0, 0))
    in_specs = [pl.BlockSpec((C_in, R, F), lambda i: (0, i, 0))]
    if has_res:
        in_specs.append(pl.BlockSpec((C_in, R, F), lambda i: (0, i, 0)))
    in_specs += [
        pl.BlockSpec((C_in * F, H), lambda i: (0, 0)),
        vec(), vec(), vec(),
        pl.BlockSpec((H, H), lambda i: (0, 0)),
        vec(), vec(), vec(),
        vec(), vec(),
    ]
    if is_last:
        out_specs = pl.BlockSpec((1, H), lambda i: (0, 0))
        out_shape = jax.ShapeDtypeStruct((1, H), jnp.float32)
    else:
        out_specs = pl.BlockSpec((C_out, R, F), lambda i: (0, i, 0))
        out_shape = jax.ShapeDtypeStruct((C_out, NP, F), jnp.float32)
    return pl.pallas_call(
        body, grid=grid, in_specs=in_specs, out_specs=out_specs,
        out_shape=out_shape)


def kernel(h, edge_index, params):
    N, D = h.shape
    E = edge_index.shape[1]
    layers = params['layers']
    H = layers[0]['w1'].shape[1]
    L = len(layers)
    Cmax = H // F

    NP = pl.cdiv(N, NS * 8) * NS * 8
    EB = NS * F * 8
    EP = pl.cdiv(E, EB) * EB
    src = edge_index[0].astype(jnp.int32)
    dst = edge_index[1].astype(jnp.int32)
    src_p = jnp.concatenate([src, jnp.zeros((EP - E,), jnp.int32)])
    dst_p = jnp.concatenate([dst, jnp.full((EP - E,), NP, jnp.int32)])
    srcC = (src_p[None, :]
            + (jnp.arange(Cmax, dtype=jnp.int32) * NP)[:, None]
            ).reshape(Cmax, EP // F, F)
    dst2 = dst_p.reshape(EP // F, F)

    R = 1024
    x = _make_input_ln(N, NP, D, R)(h, params['in_g'][None, :],
                                    params['in_b'][None, :])
    for i in range(L):
        C_in = x.shape[0]
        seg = _make_segsum(C_in, NP, EP)
        z = seg(x.reshape(C_in * NP, F), srcC[:C_in], dst2)
        p = layers[i]
        mlp = _make_mlp(N, NP, C_in, H, R, has_res=(i > 0), is_last=(i == L - 1))
        args = [z] + ([x] if i > 0 else []) + [
            p['w1'], p['b1'][None, :], p['ln1_g'][None, :], p['ln1_b'][None, :],
            p['w2'], p['b2'][None, :], p['ln2_g'][None, :], p['ln2_b'][None, :],
            p['n_g'][None, :], p['n_b'][None, :],
        ]
        x = mlp(*args)
    return x
